# baseline (speedup 1.0000x reference)
import functools
import jax
import jax.numpy as jnp
from jax import lax
from jax.experimental import pallas as pl
from jax.experimental.pallas import tpu as pltpu
from jax.experimental.pallas import tpu_sc as plsc

_E = 16
_D_MODEL = 768
_D_FF = 2048
_T = 128
_F = 512
_NF = _D_FF // _F


def _sc_row_gather(table, idx, n_rows, d):
    info = plsc.get_sparse_core_info()
    nc, ns = info.num_cores, info.num_subcores
    nw = nc * ns
    b_per_w = n_rows // nw
    mesh = plsc.VectorSubcoreMesh(core_axis_name="c", subcore_axis_name="s")

    @functools.partial(
        pl.kernel,
        mesh=mesh,
        out_type=jax.ShapeDtypeStruct((n_rows, d), jnp.float32),
        scratch_types=[
            pltpu.VMEM((b_per_w,), jnp.int32),
            pltpu.VMEM((b_per_w, d), jnp.float32),
            pltpu.SemaphoreType.DMA,
        ],
    )
    def k(table_hbm, idx_hbm, out_hbm, idx_v, rows_v, sem):
        wid = lax.axis_index("s") * nc + lax.axis_index("c")
        base = wid * b_per_w
        pltpu.sync_copy(idx_hbm.at[pl.ds(base, b_per_w)], idx_v)
        pltpu.async_copy(table_hbm.at[idx_v], rows_v, sem).wait()
        pltpu.sync_copy(rows_v, out_hbm.at[pl.ds(base, b_per_w)])

    return k(table, idx)


def _gate_body(x_ref, wg_ref, eid_ref):
    logits = jnp.dot(x_ref[...], wg_ref[...], preferred_element_type=jnp.float32)
    m = jnp.max(logits, axis=-1, keepdims=True)
    s = jnp.exp(logits - m)
    p = s / jnp.sum(s, axis=-1, keepdims=True)
    eid_ref[0, 0, :] = jnp.argmax(p, axis=-1).astype(jnp.int32)


def _gate(flat, Wg):
    n = flat.shape[0]
    nblk = n // _T
    eid2d = pl.pallas_call(
        _gate_body,
        grid=(nblk,),
        in_specs=[
            pl.BlockSpec((_T, _D_MODEL), lambda i: (i, 0)),
            pl.BlockSpec((_D_MODEL, _E), lambda i: (0, 0)),
        ],
        out_specs=pl.BlockSpec((1, 1, _T), lambda i: (i, 0, 0)),
        out_shape=jax.ShapeDtypeStruct((nblk, 1, _T), jnp.int32),
    )(flat, Wg)
    return eid2d.reshape(-1)


def _moe_body(eids_ref, meta_ref, x_ref, w1_ref, b1_ref, w2_ref, b2_ref, o_ref):
    g = pl.program_id(0)
    f = pl.program_id(1)

    @pl.when(f == 0)
    def _():
        o_ref[...] = jnp.zeros_like(o_ref)

    @pl.when(g < meta_ref[0])
    def _():
        h = jnp.dot(x_ref[...], w1_ref[0], preferred_element_type=jnp.float32)
        h = jnp.maximum(h + b1_ref[0], 0.0)
        o_ref[...] += jnp.dot(h, w2_ref[0], preferred_element_type=jnp.float32)

    @pl.when(jnp.logical_and(f == _NF - 1, g < meta_ref[0]))
    def _():
        o_ref[...] += b2_ref[0]


def _grouped_mlp(x_pad, tile_eid, ntiles, W1, b1, W2, b2, g_max):
    grid_spec = pltpu.PrefetchScalarGridSpec(
        num_scalar_prefetch=2,
        grid=(g_max, _NF),
        in_specs=[
            pl.BlockSpec((_T, _D_MODEL), lambda g, f, e, m: (g, 0)),
            pl.BlockSpec((1, _D_MODEL, _F), lambda g, f, e, m: (e[g], 0, f)),
            pl.BlockSpec((1, 1, _F), lambda g, f, e, m: (e[g], 0, f)),
            pl.BlockSpec((1, _F, _D_MODEL), lambda g, f, e, m: (e[g], f, 0)),
            pl.BlockSpec((1, 1, _D_MODEL), lambda g, f, e, m: (e[g], 0, 0)),
        ],
        out_specs=pl.BlockSpec((_T, _D_MODEL), lambda g, f, e, m: (g, 0)),
    )
    return pl.pallas_call(
        _moe_body,
        grid_spec=grid_spec,
        out_shape=jax.ShapeDtypeStruct((g_max * _T, _D_MODEL), jnp.float32),
        compiler_params=pltpu.CompilerParams(
            dimension_semantics=("arbitrary", "arbitrary"),
        ),
    )(tile_eid, ntiles, x_pad, W1, b1.reshape(_E, 1, _D_FF), W2,
      b2.reshape(_E, 1, _D_MODEL))


def kernel(inputs, Wg, W1, b1, W2, b2):
    flat = inputs.reshape((-1, inputs.shape[-1]))
    n = flat.shape[0]
    g_max = n // _T + _E

    eid = _gate(flat, Wg)

    counts = jnp.bincount(eid, length=_E)
    start = jnp.concatenate([jnp.zeros((1,), jnp.int32),
                             jnp.cumsum(counts)[:-1].astype(jnp.int32)])
    order = jnp.argsort(eid, stable=True).astype(jnp.int32)
    inv = jnp.zeros((n,), jnp.int32).at[order].set(jnp.arange(n, dtype=jnp.int32))

    tiles_e = (counts + _T - 1) // _T
    cum_tiles = jnp.cumsum(tiles_e).astype(jnp.int32)
    tile_off = cum_tiles - tiles_e.astype(jnp.int32)
    ntiles = cum_tiles[-1]

    gidx = jnp.arange(g_max, dtype=jnp.int32)
    tile_eid = jnp.minimum(
        jnp.sum(gidx[:, None] >= cum_tiles[None, :], axis=1), _E - 1
    ).astype(jnp.int32)

    p = jnp.arange(g_max * _T, dtype=jnp.int32)
    pg = p // _T
    pe = tile_eid[pg]
    rank = (pg - tile_off[pe]) * _T + (p % _T)
    valid = rank < counts[pe]
    src = jnp.where(valid, order[jnp.minimum(start[pe] + rank, n - 1)], 0)

    te = eid
    trank = inv - start[te]
    pos = (tile_off[te] + trank // _T) * _T + trank % _T

    x_pad = _sc_row_gather(flat, src, g_max * _T, _D_MODEL)
    y_pad = _grouped_mlp(x_pad, tile_eid, jnp.array([ntiles], jnp.int32),
                         W1, b1, W2, b2, g_max)
    out = _sc_row_gather(y_pad, pos, n, _D_MODEL)
    return out.reshape(inputs.shape[:-1] + (_D_MODEL,))


# speedup vs baseline: 1.2008x; 1.2008x over previous
import functools
import jax
import jax.numpy as jnp
from jax import lax
from jax.experimental import pallas as pl
from jax.experimental.pallas import tpu as pltpu
from jax.experimental.pallas import tpu_sc as plsc

_E = 16
_D_MODEL = 768
_D_FF = 2048
_T = 128
_F = 512
_NF = _D_FF // _F


def _sc_row_gather(table, idx, n_rows, d):
    info = plsc.get_sparse_core_info()
    nc, ns = info.num_cores, info.num_subcores
    nw = nc * ns
    b_per_w = n_rows // nw
    mesh = plsc.VectorSubcoreMesh(core_axis_name="c", subcore_axis_name="s")

    @functools.partial(
        pl.kernel,
        mesh=mesh,
        out_type=jax.ShapeDtypeStruct((n_rows, d), jnp.float32),
        scratch_types=[
            pltpu.VMEM((b_per_w,), jnp.int32),
            pltpu.VMEM((b_per_w, d), jnp.float32),
            pltpu.SemaphoreType.DMA,
        ],
    )
    def k(table_hbm, idx_hbm, out_hbm, idx_v, rows_v, sem):
        wid = lax.axis_index("s") * nc + lax.axis_index("c")
        base = wid * b_per_w
        pltpu.sync_copy(idx_hbm.at[pl.ds(base, b_per_w)], idx_v)
        pltpu.async_copy(table_hbm.at[idx_v], rows_v, sem).wait()
        pltpu.sync_copy(rows_v, out_hbm.at[pl.ds(base, b_per_w)])

    return k(table, idx)


def _gate_body(x_ref, wg_ref, eid_ref):
    logits = jnp.dot(x_ref[...], wg_ref[...], preferred_element_type=jnp.float32)
    m = jnp.max(logits, axis=-1, keepdims=True)
    s = jnp.exp(logits - m)
    p = s / jnp.sum(s, axis=-1, keepdims=True)
    eid_ref[0, 0, :] = jnp.argmax(p, axis=-1).astype(jnp.int32)


def _gate(flat, Wg):
    n = flat.shape[0]
    nblk = n // _T
    eid2d = pl.pallas_call(
        _gate_body,
        grid=(nblk,),
        in_specs=[
            pl.BlockSpec((_T, _D_MODEL), lambda i: (i, 0)),
            pl.BlockSpec((_D_MODEL, _E), lambda i: (0, 0)),
        ],
        out_specs=pl.BlockSpec((1, 1, _T), lambda i: (i, 0, 0)),
        out_shape=jax.ShapeDtypeStruct((nblk, 1, _T), jnp.int32),
    )(flat, Wg)
    return eid2d.reshape(-1)


def _moe_body(eids_ref, meta_ref, x_ref, w1_ref, b1_ref, w2_ref, b2_ref, o_ref):
    g = pl.program_id(0)
    f = pl.program_id(1)

    @pl.when(f == 0)
    def _():
        o_ref[...] = jnp.zeros_like(o_ref)

    @pl.when(g < meta_ref[0])
    def _():
        h = jnp.dot(x_ref[...], w1_ref[0], preferred_element_type=jnp.float32)
        h = jnp.maximum(h + b1_ref[0], 0.0)
        o_ref[...] += jnp.dot(h, w2_ref[0], preferred_element_type=jnp.float32)

    @pl.when(jnp.logical_and(f == _NF - 1, g < meta_ref[0]))
    def _():
        o_ref[...] += b2_ref[0]


def _grouped_mlp(x_pad, tile_eid, ntiles, W1, b1, W2, b2, g_max):
    grid_spec = pltpu.PrefetchScalarGridSpec(
        num_scalar_prefetch=2,
        grid=(g_max, _NF),
        in_specs=[
            pl.BlockSpec((_T, _D_MODEL), lambda g, f, e, m: (g, 0)),
            pl.BlockSpec((1, _D_MODEL, _F), lambda g, f, e, m: (e[g], 0, f)),
            pl.BlockSpec((1, 1, _F), lambda g, f, e, m: (e[g], 0, f)),
            pl.BlockSpec((1, _F, _D_MODEL), lambda g, f, e, m: (e[g], f, 0)),
            pl.BlockSpec((1, 1, _D_MODEL), lambda g, f, e, m: (e[g], 0, 0)),
        ],
        out_specs=pl.BlockSpec((_T, _D_MODEL), lambda g, f, e, m: (g, 0)),
    )
    return pl.pallas_call(
        _moe_body,
        grid_spec=grid_spec,
        out_shape=jax.ShapeDtypeStruct((g_max * _T, _D_MODEL), jnp.float32),
        compiler_params=pltpu.CompilerParams(
            dimension_semantics=("arbitrary", "arbitrary"),
        ),
    )(tile_eid, ntiles, x_pad, W1, b1.reshape(_E, 1, _D_FF), W2,
      b2.reshape(_E, 1, _D_MODEL))


def kernel(inputs, Wg, W1, b1, W2, b2):
    flat = inputs.reshape((-1, inputs.shape[-1]))
    n = flat.shape[0]
    g_max = n // _T + _E

    eid = _gate(flat, Wg)

    onehot = (eid[:, None] == jnp.arange(_E, dtype=jnp.int32)[None, :])
    cum = jnp.cumsum(onehot.astype(jnp.int32), axis=0)
    counts = cum[-1]
    start = jnp.concatenate([jnp.zeros((1,), jnp.int32),
                             jnp.cumsum(counts)[:-1].astype(jnp.int32)])
    rank_tok = jnp.take_along_axis(cum, eid[:, None], axis=1)[:, 0] - 1
    inv = start[eid] + rank_tok
    order = jnp.zeros((n,), jnp.int32).at[inv].set(jnp.arange(n, dtype=jnp.int32))

    tiles_e = (counts + _T - 1) // _T
    cum_tiles = jnp.cumsum(tiles_e).astype(jnp.int32)
    tile_off = cum_tiles - tiles_e.astype(jnp.int32)
    ntiles = cum_tiles[-1]

    gidx = jnp.arange(g_max, dtype=jnp.int32)
    tile_eid = jnp.minimum(
        jnp.sum(gidx[:, None] >= cum_tiles[None, :], axis=1), _E - 1
    ).astype(jnp.int32)

    p = jnp.arange(g_max * _T, dtype=jnp.int32)
    pg = p // _T
    pe = tile_eid[pg]
    rank = (pg - tile_off[pe]) * _T + (p % _T)
    valid = rank < counts[pe]
    src = jnp.where(valid, order[jnp.minimum(start[pe] + rank, n - 1)],
                    p % n)

    te = eid
    trank = inv - start[te]
    pos = (tile_off[te] + trank // _T) * _T + trank % _T

    x_pad = _sc_row_gather(flat, src, g_max * _T, _D_MODEL)
    y_pad = _grouped_mlp(x_pad, tile_eid, jnp.array([ntiles], jnp.int32),
                         W1, b1, W2, b2, g_max)
    out = _sc_row_gather(y_pad, pos, n, _D_MODEL)
    return out.reshape(inputs.shape[:-1] + (_D_MODEL,))


# speedup vs baseline: 1.2942x; 1.0778x over previous
import functools
import jax
import jax.numpy as jnp
from jax import lax
from jax.experimental import pallas as pl
from jax.experimental.pallas import tpu as pltpu
from jax.experimental.pallas import tpu_sc as plsc

_E = 16
_D_MODEL = 768
_D_FF = 2048
_T = 128
_F = 512
_NF = _D_FF // _F


def _sc_row_gather(table, idx, n_rows, d):
    info = plsc.get_sparse_core_info()
    nc, ns = info.num_cores, info.num_subcores
    nw = nc * ns
    b_per_w = n_rows // nw
    mesh = plsc.VectorSubcoreMesh(core_axis_name="c", subcore_axis_name="s")

    @functools.partial(
        pl.kernel,
        mesh=mesh,
        out_type=jax.ShapeDtypeStruct((n_rows, d), jnp.float32),
        scratch_types=[
            pltpu.VMEM((b_per_w,), jnp.int32),
            pltpu.VMEM((b_per_w, d), jnp.float32),
            pltpu.SemaphoreType.DMA,
        ],
    )
    def k(table_hbm, idx_hbm, out_hbm, idx_v, rows_v, sem):
        wid = lax.axis_index("s") * nc + lax.axis_index("c")
        base = wid * b_per_w
        pltpu.sync_copy(idx_hbm.at[pl.ds(base, b_per_w)], idx_v)
        pltpu.async_copy(table_hbm.at[idx_v], rows_v, sem).wait()
        pltpu.sync_copy(rows_v, out_hbm.at[pl.ds(base, b_per_w)])

    return k(table, idx)


def _gate_body(x_ref, wg_ref, eid_ref):
    logits = jnp.dot(x_ref[...], wg_ref[...], preferred_element_type=jnp.float32)
    m = jnp.max(logits, axis=-1, keepdims=True)
    s = jnp.exp(logits - m)
    p = s / jnp.sum(s, axis=-1, keepdims=True)
    eid_ref[0, 0, :] = jnp.argmax(p, axis=-1).astype(jnp.int32)


def _gate(flat, Wg):
    n = flat.shape[0]
    nblk = n // _T
    eid2d = pl.pallas_call(
        _gate_body,
        grid=(nblk,),
        in_specs=[
            pl.BlockSpec((_T, _D_MODEL), lambda i: (i, 0)),
            pl.BlockSpec((_D_MODEL, _E), lambda i: (0, 0)),
        ],
        out_specs=pl.BlockSpec((1, 1, _T), lambda i: (i, 0, 0)),
        out_shape=jax.ShapeDtypeStruct((nblk, 1, _T), jnp.int32),
    )(flat, Wg)
    return eid2d.reshape(-1)


def _moe_body(eids_ref, meta_ref, x_ref, w1_ref, b1_ref, w2_ref, b2_ref, o_ref):
    g = pl.program_id(0)
    f = pl.program_id(1)

    @pl.when(f == 0)
    def _():
        o_ref[...] = jnp.zeros_like(o_ref)

    @pl.when(g < meta_ref[0])
    def _():
        h = jnp.dot(x_ref[...], w1_ref[0], preferred_element_type=jnp.float32)
        h = jnp.maximum(h + b1_ref[0], 0.0)
        o_ref[...] += jnp.dot(h, w2_ref[0], preferred_element_type=jnp.float32)

    @pl.when(jnp.logical_and(f == _NF - 1, g < meta_ref[0]))
    def _():
        o_ref[...] += b2_ref[0]


def _grouped_mlp(x_pad, tile_eid, ntiles, W1, b1, W2, b2, g_max):
    grid_spec = pltpu.PrefetchScalarGridSpec(
        num_scalar_prefetch=2,
        grid=(g_max, _NF),
        in_specs=[
            pl.BlockSpec((_T, _D_MODEL), lambda g, f, e, m: (g, 0)),
            pl.BlockSpec((1, _D_MODEL, _F),
                         lambda g, f, e, m: (e[g], 0,
                                             jnp.where(g < m[0], f, _NF - 1))),
            pl.BlockSpec((1, 1, _F),
                         lambda g, f, e, m: (e[g], 0,
                                             jnp.where(g < m[0], f, _NF - 1))),
            pl.BlockSpec((1, _F, _D_MODEL),
                         lambda g, f, e, m: (e[g],
                                             jnp.where(g < m[0], f, _NF - 1),
                                             0)),
            pl.BlockSpec((1, 1, _D_MODEL), lambda g, f, e, m: (e[g], 0, 0)),
        ],
        out_specs=pl.BlockSpec((_T, _D_MODEL), lambda g, f, e, m: (g, 0)),
    )
    return pl.pallas_call(
        _moe_body,
        grid_spec=grid_spec,
        out_shape=jax.ShapeDtypeStruct((g_max * _T, _D_MODEL), jnp.float32),
        compiler_params=pltpu.CompilerParams(
            dimension_semantics=("arbitrary", "arbitrary"),
        ),
    )(tile_eid, ntiles, x_pad, W1, b1.reshape(_E, 1, _D_FF), W2,
      b2.reshape(_E, 1, _D_MODEL))


def kernel(inputs, Wg, W1, b1, W2, b2):
    flat = inputs.reshape((-1, inputs.shape[-1]))
    n = flat.shape[0]
    g_max = n // _T + _E

    eid = _gate(flat, Wg)

    onehot = (eid[:, None] == jnp.arange(_E, dtype=jnp.int32)[None, :])
    cum = jnp.cumsum(onehot.astype(jnp.int32), axis=0)
    counts = cum[-1]
    start = jnp.concatenate([jnp.zeros((1,), jnp.int32),
                             jnp.cumsum(counts)[:-1].astype(jnp.int32)])
    rank_tok = jnp.sum(cum * onehot.astype(jnp.int32), axis=1) - 1
    inv = start[eid] + rank_tok
    order = jnp.zeros((n,), jnp.int32).at[inv].set(jnp.arange(n, dtype=jnp.int32))

    tiles_e = (counts + _T - 1) // _T
    cum_tiles = jnp.cumsum(tiles_e).astype(jnp.int32)
    tile_off = cum_tiles - tiles_e.astype(jnp.int32)
    ntiles = cum_tiles[-1]

    gidx = jnp.arange(g_max, dtype=jnp.int32)
    tile_eid = jnp.minimum(
        jnp.sum(gidx[:, None] >= cum_tiles[None, :], axis=1), _E - 1
    ).astype(jnp.int32)

    p = jnp.arange(g_max * _T, dtype=jnp.int32)
    pg = p // _T
    pe = tile_eid[pg]
    rank = (pg - tile_off[pe]) * _T + (p % _T)
    valid = rank < counts[pe]
    src = jnp.where(valid, order[jnp.minimum(start[pe] + rank, n - 1)],
                    p % n)

    te = eid
    trank = inv - start[te]
    pos = (tile_off[te] + trank // _T) * _T + trank % _T

    x_pad = _sc_row_gather(flat, src, g_max * _T, _D_MODEL)
    y_pad = _grouped_mlp(x_pad, tile_eid, jnp.array([ntiles], jnp.int32),
                         W1, b1, W2, b2, g_max)
    out = _sc_row_gather(y_pad, pos, n, _D_MODEL)
    return out.reshape(inputs.shape[:-1] + (_D_MODEL,))


# speedup vs baseline: 2.7752x; 2.1443x over previous
import functools
import jax
import jax.numpy as jnp
from jax import lax
from jax.experimental import pallas as pl
from jax.experimental.pallas import tpu as pltpu
from jax.experimental.pallas import tpu_sc as plsc

_E = 16
_D_MODEL = 768
_D_FF = 2048
_T = 256
_GATE_T = 256


def _sc_row_gather(table, idx, n_rows, d):
    info = plsc.get_sparse_core_info()
    nc, ns = info.num_cores, info.num_subcores
    nw = nc * ns
    b_per_w = n_rows // nw
    chunk = 64
    n_chunks = b_per_w // chunk
    assert chunk * n_chunks == b_per_w
    nbuf = min(2, n_chunks)
    mesh = plsc.VectorSubcoreMesh(core_axis_name="c", subcore_axis_name="s")

    @functools.partial(
        pl.kernel,
        mesh=mesh,
        out_type=jax.ShapeDtypeStruct((n_rows, d), jnp.float32),
        scratch_types=[pltpu.VMEM((chunk,), jnp.int32) for _ in range(n_chunks)]
        + [pltpu.VMEM((chunk, d), jnp.float32) for _ in range(nbuf)]
        + [pltpu.SemaphoreType.DMA for _ in range(nbuf)],
    )
    def k(table_hbm, idx_hbm, out_hbm, *rest):
        idx_bufs = rest[:n_chunks]
        rows_bufs = rest[n_chunks:n_chunks + nbuf]
        sems = rest[n_chunks + nbuf:]
        wid = lax.axis_index("s") * nc + lax.axis_index("c")
        base = wid * b_per_w
        for c in range(n_chunks):
            pltpu.sync_copy(idx_hbm.at[pl.ds(base + c * chunk, chunk)],
                            idx_bufs[c])
        copies = [None] * n_chunks
        for c in range(min(nbuf, n_chunks)):
            copies[c] = pltpu.async_copy(table_hbm.at[idx_bufs[c]],
                                         rows_bufs[c % nbuf], sems[c % nbuf])
        for c in range(n_chunks):
            copies[c].wait()
            pltpu.sync_copy(rows_bufs[c % nbuf],
                            out_hbm.at[pl.ds(base + c * chunk, chunk)])
            nxt = c + nbuf
            if nxt < n_chunks:
                copies[nxt] = pltpu.async_copy(table_hbm.at[idx_bufs[nxt]],
                                               rows_bufs[nxt % nbuf],
                                               sems[nxt % nbuf])

    return k(table, idx)


def _gate_body(x_ref, wg_ref, eid_ref):
    logits = jnp.dot(x_ref[...], wg_ref[...], preferred_element_type=jnp.float32)
    m = jnp.max(logits, axis=-1, keepdims=True)
    s = jnp.exp(logits - m)
    p = s / jnp.sum(s, axis=-1, keepdims=True)
    eid_ref[0, 0, :] = jnp.argmax(p, axis=-1).astype(jnp.int32)


def _gate(flat, Wg):
    n = flat.shape[0]
    nblk = n // _GATE_T
    eid2d = pl.pallas_call(
        _gate_body,
        grid=(nblk,),
        in_specs=[
            pl.BlockSpec((_GATE_T, _D_MODEL), lambda i: (i, 0)),
            pl.BlockSpec((_D_MODEL, _E), lambda i: (0, 0)),
        ],
        out_specs=pl.BlockSpec((1, 1, _GATE_T), lambda i: (i, 0, 0)),
        out_shape=jax.ShapeDtypeStruct((nblk, 1, _GATE_T), jnp.int32),
    )(flat, Wg)
    return eid2d.reshape(-1)


def _moe_body(eids_ref, meta_ref, x_ref, w1_ref, b1_ref, w2_ref, b2_ref, o_ref):
    g = pl.program_id(0)

    @pl.when(g < meta_ref[0])
    def _():
        h = jnp.dot(x_ref[...], w1_ref[0], preferred_element_type=jnp.float32)
        h = jnp.maximum(h + b1_ref[0], 0.0)
        o_ref[...] = (jnp.dot(h, w2_ref[0], preferred_element_type=jnp.float32)
                      + b2_ref[0])


def _grouped_mlp(x_pad, tile_eid, ntiles, W1, b1, W2, b2, g_max):
    grid_spec = pltpu.PrefetchScalarGridSpec(
        num_scalar_prefetch=2,
        grid=(g_max,),
        in_specs=[
            pl.BlockSpec((_T, _D_MODEL), lambda g, e, m: (g, 0)),
            pl.BlockSpec((1, _D_MODEL, _D_FF), lambda g, e, m: (e[g], 0, 0)),
            pl.BlockSpec((1, 1, _D_FF), lambda g, e, m: (e[g], 0, 0)),
            pl.BlockSpec((1, _D_FF, _D_MODEL), lambda g, e, m: (e[g], 0, 0)),
            pl.BlockSpec((1, 1, _D_MODEL), lambda g, e, m: (e[g], 0, 0)),
        ],
        out_specs=pl.BlockSpec((_T, _D_MODEL), lambda g, e, m: (g, 0)),
    )
    return pl.pallas_call(
        _moe_body,
        grid_spec=grid_spec,
        out_shape=jax.ShapeDtypeStruct((g_max * _T, _D_MODEL), jnp.float32),
        compiler_params=pltpu.CompilerParams(
            dimension_semantics=("arbitrary",),
        ),
    )(tile_eid, ntiles, x_pad, W1, b1.reshape(_E, 1, _D_FF), W2,
      b2.reshape(_E, 1, _D_MODEL))


def kernel(inputs, Wg, W1, b1, W2, b2):
    flat = inputs.reshape((-1, inputs.shape[-1]))
    n = flat.shape[0]
    g_max = n // _T + _E

    eid = _gate(flat, Wg)

    onehot = (eid[:, None] == jnp.arange(_E, dtype=jnp.int32)[None, :]
              ).astype(jnp.int32)
    cum = jnp.cumsum(onehot, axis=0)
    counts = cum[-1]
    start = jnp.concatenate([jnp.zeros((1,), jnp.int32),
                             jnp.cumsum(counts)[:-1].astype(jnp.int32)])
    rank_tok = jnp.sum(cum * onehot, axis=1) - 1

    tiles_e = (counts + _T - 1) // _T
    cum_tiles = jnp.cumsum(tiles_e).astype(jnp.int32)
    tile_off = (cum_tiles - tiles_e).astype(jnp.int32)
    ntiles = cum_tiles[-1]

    gidx = jnp.arange(g_max, dtype=jnp.int32)
    tile_eid = jnp.minimum(
        jnp.sum((gidx[:, None] >= cum_tiles[None, :]).astype(jnp.int32), axis=1),
        _E - 1).astype(jnp.int32)

    tile_off_tok = jnp.sum(tile_off[None, :] * onehot, axis=1)
    pos = (tile_off_tok + rank_tok // _T) * _T + rank_tok % _T

    p = jnp.arange(g_max * _T, dtype=jnp.int32)
    src = (p % n).at[pos].set(jnp.arange(n, dtype=jnp.int32))

    x_pad = _sc_row_gather(flat, src, g_max * _T, _D_MODEL)
    y_pad = _grouped_mlp(x_pad, tile_eid, jnp.array([ntiles], jnp.int32),
                         W1, b1, W2, b2, g_max)
    out = _sc_row_gather(y_pad, pos, n, _D_MODEL)
    return out.reshape(inputs.shape[:-1] + (_D_MODEL,))


# speedup vs baseline: 3.4664x; 1.2490x over previous
import functools
import jax
import jax.numpy as jnp
from jax import lax
from jax.experimental import pallas as pl
from jax.experimental.pallas import tpu as pltpu
from jax.experimental.pallas import tpu_sc as plsc

_E = 16
_D_MODEL = 768
_D_FF = 2048
_T = 256


def _sc_info():
    info = plsc.get_sparse_core_info()
    return info.num_cores, info.num_subcores


def _sc_row_gather(table, idx, n_rows, d):
    nc, ns = _sc_info()
    nw = nc * ns
    b_per_w = n_rows // nw
    chunk = 64
    n_chunks = b_per_w // chunk
    assert chunk * n_chunks == b_per_w
    nbuf = min(2, n_chunks)
    mesh = plsc.VectorSubcoreMesh(core_axis_name="c", subcore_axis_name="s")

    @functools.partial(
        pl.kernel,
        mesh=mesh,
        out_type=jax.ShapeDtypeStruct((n_rows, d), jnp.float32),
        scratch_types=[pltpu.VMEM((chunk,), jnp.int32) for _ in range(n_chunks)]
        + [pltpu.VMEM((chunk, d), jnp.float32) for _ in range(nbuf)]
        + [pltpu.SemaphoreType.DMA for _ in range(nbuf)],
    )
    def k(table_hbm, idx_hbm, out_hbm, *rest):
        idx_bufs = rest[:n_chunks]
        rows_bufs = rest[n_chunks:n_chunks + nbuf]
        sems = rest[n_chunks + nbuf:]
        wid = lax.axis_index("s") * nc + lax.axis_index("c")
        base = wid * b_per_w
        for c in range(n_chunks):
            pltpu.sync_copy(idx_hbm.at[pl.ds(base + c * chunk, chunk)],
                            idx_bufs[c])
        copies = [None] * n_chunks
        for c in range(min(nbuf, n_chunks)):
            copies[c] = pltpu.async_copy(table_hbm.at[idx_bufs[c]],
                                         rows_bufs[c % nbuf], sems[c % nbuf])
        for c in range(n_chunks):
            copies[c].wait()
            pltpu.sync_copy(rows_bufs[c % nbuf],
                            out_hbm.at[pl.ds(base + c * chunk, chunk)])
            nxt = c + nbuf
            if nxt < n_chunks:
                copies[nxt] = pltpu.async_copy(table_hbm.at[idx_bufs[nxt]],
                                               rows_bufs[nxt % nbuf],
                                               sems[nxt % nbuf])

    return k(table, idx)


def _sc_row_scatter(rows, idx, n_out_rows, d):
    nc, ns = _sc_info()
    nw = nc * ns
    n_in = rows.shape[0]
    b_per_w = n_in // nw
    chunk = 64
    n_chunks = b_per_w // chunk
    assert chunk * n_chunks == b_per_w
    mesh = plsc.VectorSubcoreMesh(core_axis_name="c", subcore_axis_name="s")

    @functools.partial(
        pl.kernel,
        mesh=mesh,
        out_type=jax.ShapeDtypeStruct((n_out_rows, d), jnp.float32),
        scratch_types=[pltpu.VMEM((chunk,), jnp.int32) for _ in range(n_chunks)]
        + [pltpu.VMEM((chunk, d), jnp.float32) for _ in range(n_chunks)]
        + [pltpu.SemaphoreType.DMA],
    )
    def k(rows_hbm, idx_hbm, out_hbm, *rest):
        idx_bufs = rest[:n_chunks]
        rows_bufs = rest[n_chunks:2 * n_chunks]
        sem = rest[2 * n_chunks]
        wid = lax.axis_index("s") * nc + lax.axis_index("c")
        base = wid * b_per_w
        for c in range(n_chunks):
            pltpu.sync_copy(idx_hbm.at[pl.ds(base + c * chunk, chunk)],
                            idx_bufs[c])
            pltpu.sync_copy(rows_hbm.at[pl.ds(base + c * chunk, chunk)],
                            rows_bufs[c])
        copies = [pltpu.async_copy(rows_bufs[c], out_hbm.at[idx_bufs[c]], sem)
                  for c in range(n_chunks)]
        for cp in copies:
            cp.wait()

    return k(rows, idx)


def _route_body(x_ref, wg_ref, pos_ref, meta_ref):
    n = x_ref.shape[0]
    g_max = meta_ref.shape[0] - 8
    logits = jnp.dot(x_ref[...], wg_ref[...], preferred_element_type=jnp.float32)
    m = jnp.max(logits, axis=-1, keepdims=True)
    s = jnp.exp(logits - m)
    prob = s / jnp.sum(s, axis=-1, keepdims=True)
    eid = jnp.argmax(prob, axis=-1).astype(jnp.int32)

    oh = (eid[:, None] == lax.broadcasted_iota(jnp.int32, (1, _E), 1)
          ).astype(jnp.int32)
    cum = oh
    sh = 1
    while sh < n:
        cum = cum + jnp.concatenate(
            [jnp.zeros((sh, _E), jnp.int32), cum[:-sh]], axis=0)
        sh *= 2
    counts = cum[-1:].astype(jnp.float32)

    col = lax.broadcasted_iota(jnp.int32, (_E, _E), 1)
    row = lax.broadcasted_iota(jnp.int32, (_E, _E), 0)
    tri_excl = (row < col).astype(jnp.float32)
    tri_incl = (row <= col).astype(jnp.float32)

    start = jnp.dot(counts, tri_excl,
                    preferred_element_type=jnp.float32).astype(jnp.int32)
    tiles_e = jnp.floor((counts + (_T - 1)) * (1.0 / _T))
    cum_tiles = jnp.dot(tiles_e, tri_incl,
                        preferred_element_type=jnp.float32).astype(jnp.int32)
    tile_off = cum_tiles - tiles_e.astype(jnp.int32)
    ntiles = cum_tiles[0, _E - 1]

    rank = jnp.sum(cum * oh, axis=1) - 1
    start_tok = jnp.sum(start * oh, axis=1)
    toff_tok = jnp.sum(tile_off * oh, axis=1)
    del start_tok
    pos_ref[...] = (toff_tok + rank // _T) * _T + rank % _T

    gi = lax.broadcasted_iota(jnp.int32, (g_max, _E), 0)
    tile_eid = jnp.minimum(
        jnp.sum((gi >= cum_tiles).astype(jnp.int32), axis=1), _E - 1)
    meta_ref[...] = jnp.concatenate(
        [tile_eid, jnp.full((8,), ntiles, jnp.int32)], axis=0)


def _route(flat, Wg, g_max):
    n = flat.shape[0]
    return pl.pallas_call(
        _route_body,
        out_shape=[jax.ShapeDtypeStruct((n,), jnp.int32),
                   jax.ShapeDtypeStruct((g_max + 8,), jnp.int32)],
    )(flat, Wg)


def _moe_body(eids_ref, meta_ref, x_ref, w1_ref, b1_ref, w2_ref, b2_ref, o_ref):
    g = pl.program_id(0)

    @pl.when(g < meta_ref[0])
    def _():
        h = jnp.dot(x_ref[...], w1_ref[0], preferred_element_type=jnp.float32)
        h = jnp.maximum(h + b1_ref[0], 0.0)
        o_ref[...] = (jnp.dot(h, w2_ref[0], preferred_element_type=jnp.float32)
                      + b2_ref[0])


def _grouped_mlp(x_pad, tile_eid, ntiles, W1, b1, W2, b2, g_max):
    grid_spec = pltpu.PrefetchScalarGridSpec(
        num_scalar_prefetch=2,
        grid=(g_max,),
        in_specs=[
            pl.BlockSpec((_T, _D_MODEL), lambda g, e, m: (g, 0)),
            pl.BlockSpec((1, _D_MODEL, _D_FF), lambda g, e, m: (e[g], 0, 0)),
            pl.BlockSpec((1, 1, _D_FF), lambda g, e, m: (e[g], 0, 0)),
            pl.BlockSpec((1, _D_FF, _D_MODEL), lambda g, e, m: (e[g], 0, 0)),
            pl.BlockSpec((1, 1, _D_MODEL), lambda g, e, m: (e[g], 0, 0)),
        ],
        out_specs=pl.BlockSpec((_T, _D_MODEL), lambda g, e, m: (g, 0)),
    )
    return pl.pallas_call(
        _moe_body,
        grid_spec=grid_spec,
        out_shape=jax.ShapeDtypeStruct((g_max * _T, _D_MODEL), jnp.float32),
        compiler_params=pltpu.CompilerParams(
            dimension_semantics=("arbitrary",),
        ),
    )(tile_eid, ntiles, x_pad, W1, b1.reshape(_E, 1, _D_FF), W2,
      b2.reshape(_E, 1, _D_MODEL))


def kernel(inputs, Wg, W1, b1, W2, b2):
    flat = inputs.reshape((-1, inputs.shape[-1]))
    n = flat.shape[0]
    g_max = n // _T + _E

    pos, meta = _route(flat, Wg, g_max)
    tile_eid = meta[:g_max]
    ntiles = meta[g_max:g_max + 1]

    x_pad = _sc_row_scatter(flat, pos, g_max * _T, _D_MODEL)
    y_pad = _grouped_mlp(x_pad, tile_eid, ntiles, W1, b1, W2, b2, g_max)
    out = _sc_row_gather(y_pad, pos, n, _D_MODEL)
    return out.reshape(inputs.shape[:-1] + (_D_MODEL,))


# speedup vs baseline: 3.5100x; 1.0126x over previous
import functools
import jax
import jax.numpy as jnp
from jax import lax
from jax.experimental import pallas as pl
from jax.experimental.pallas import tpu as pltpu
from jax.experimental.pallas import tpu_sc as plsc

_E = 16
_D_MODEL = 768
_D_FF = 2048
_T = 160


def _sc_info():
    info = plsc.get_sparse_core_info()
    return info.num_cores, info.num_subcores


def _sc_row_gather(table, idx, n_rows, d):
    nc, ns = _sc_info()
    nw = nc * ns
    b_per_w = n_rows // nw
    chunk = 64
    n_chunks = b_per_w // chunk
    assert chunk * n_chunks == b_per_w
    nbuf = min(2, n_chunks)
    mesh = plsc.VectorSubcoreMesh(core_axis_name="c", subcore_axis_name="s")

    @functools.partial(
        pl.kernel,
        mesh=mesh,
        out_type=jax.ShapeDtypeStruct((n_rows, d), jnp.float32),
        scratch_types=[pltpu.VMEM((chunk,), jnp.int32) for _ in range(n_chunks)]
        + [pltpu.VMEM((chunk, d), jnp.float32) for _ in range(nbuf)]
        + [pltpu.SemaphoreType.DMA for _ in range(nbuf)],
    )
    def k(table_hbm, idx_hbm, out_hbm, *rest):
        idx_bufs = rest[:n_chunks]
        rows_bufs = rest[n_chunks:n_chunks + nbuf]
        sems = rest[n_chunks + nbuf:]
        wid = lax.axis_index("s") * nc + lax.axis_index("c")
        base = wid * b_per_w
        for c in range(n_chunks):
            pltpu.sync_copy(idx_hbm.at[pl.ds(base + c * chunk, chunk)],
                            idx_bufs[c])
        copies = [None] * n_chunks
        for c in range(min(nbuf, n_chunks)):
            copies[c] = pltpu.async_copy(table_hbm.at[idx_bufs[c]],
                                         rows_bufs[c % nbuf], sems[c % nbuf])
        for c in range(n_chunks):
            copies[c].wait()
            pltpu.sync_copy(rows_bufs[c % nbuf],
                            out_hbm.at[pl.ds(base + c * chunk, chunk)])
            nxt = c + nbuf
            if nxt < n_chunks:
                copies[nxt] = pltpu.async_copy(table_hbm.at[idx_bufs[nxt]],
                                               rows_bufs[nxt % nbuf],
                                               sems[nxt % nbuf])

    return k(table, idx)


def _sc_row_scatter(rows, idx, n_out_rows, d):
    nc, ns = _sc_info()
    nw = nc * ns
    n_in = rows.shape[0]
    b_per_w = n_in // nw
    chunk = 64
    n_chunks = b_per_w // chunk
    assert chunk * n_chunks == b_per_w
    mesh = plsc.VectorSubcoreMesh(core_axis_name="c", subcore_axis_name="s")

    @functools.partial(
        pl.kernel,
        mesh=mesh,
        out_type=jax.ShapeDtypeStruct((n_out_rows, d), jnp.float32),
        scratch_types=[pltpu.VMEM((chunk,), jnp.int32) for _ in range(n_chunks)]
        + [pltpu.VMEM((chunk, d), jnp.float32) for _ in range(n_chunks)]
        + [pltpu.SemaphoreType.DMA],
    )
    def k(rows_hbm, idx_hbm, out_hbm, *rest):
        idx_bufs = rest[:n_chunks]
        rows_bufs = rest[n_chunks:2 * n_chunks]
        sem = rest[2 * n_chunks]
        wid = lax.axis_index("s") * nc + lax.axis_index("c")
        base = wid * b_per_w
        for c in range(n_chunks):
            pltpu.sync_copy(idx_hbm.at[pl.ds(base + c * chunk, chunk)],
                            idx_bufs[c])
            pltpu.sync_copy(rows_hbm.at[pl.ds(base + c * chunk, chunk)],
                            rows_bufs[c])
        copies = [pltpu.async_copy(rows_bufs[c], out_hbm.at[idx_bufs[c]], sem)
                  for c in range(n_chunks)]
        for cp in copies:
            cp.wait()

    return k(rows, idx)


def _route_body(x_ref, wg_ref, pos_ref, meta_ref):
    n = x_ref.shape[0]
    g_max = meta_ref.shape[0] - 8
    logits = jnp.dot(x_ref[...], wg_ref[...], preferred_element_type=jnp.float32)
    m = jnp.max(logits, axis=-1, keepdims=True)
    s = jnp.exp(logits - m)
    prob = s / jnp.sum(s, axis=-1, keepdims=True)
    eid = jnp.argmax(prob, axis=-1).astype(jnp.int32)

    oh = (eid[:, None] == lax.broadcasted_iota(jnp.int32, (1, _E), 1)
          ).astype(jnp.int32)
    cum = oh
    sh = 1
    while sh < n:
        cum = cum + jnp.concatenate(
            [jnp.zeros((sh, _E), jnp.int32), cum[:-sh]], axis=0)
        sh *= 2
    counts_i = cum[-1:]
    counts = counts_i.astype(jnp.float32)

    col = lax.broadcasted_iota(jnp.int32, (_E, _E), 1)
    row = lax.broadcasted_iota(jnp.int32, (_E, _E), 0)
    tri_excl = (row < col).astype(jnp.float32)
    tri_incl = (row <= col).astype(jnp.float32)

    start = jnp.dot(counts, tri_excl,
                    preferred_element_type=jnp.float32).astype(jnp.int32)
    tiles_e = ((counts_i + (_T - 1)) // _T).astype(jnp.float32)
    cum_tiles = jnp.dot(tiles_e, tri_incl,
                        preferred_element_type=jnp.float32).astype(jnp.int32)
    tile_off = cum_tiles - tiles_e.astype(jnp.int32)
    ntiles = cum_tiles[0, _E - 1]

    rank = jnp.sum(cum * oh, axis=1) - 1
    start_tok = jnp.sum(start * oh, axis=1)
    toff_tok = jnp.sum(tile_off * oh, axis=1)
    del start_tok
    pos_ref[...] = (toff_tok + rank // _T) * _T + rank % _T

    gi = lax.broadcasted_iota(jnp.int32, (g_max, _E), 0)
    tile_eid = jnp.minimum(
        jnp.sum((gi >= cum_tiles).astype(jnp.int32), axis=1), _E - 1)
    meta_ref[...] = jnp.concatenate(
        [tile_eid, jnp.full((8,), ntiles, jnp.int32)], axis=0)


def _route(flat, Wg, g_max):
    n = flat.shape[0]
    return pl.pallas_call(
        _route_body,
        out_shape=[jax.ShapeDtypeStruct((n,), jnp.int32),
                   jax.ShapeDtypeStruct((g_max + 8,), jnp.int32)],
    )(flat, Wg)


def _moe_body(eids_ref, meta_ref, x_ref, w1_ref, b1_ref, w2_ref, b2_ref, o_ref):
    g = pl.program_id(0)

    @pl.when(g < meta_ref[0])
    def _():
        h = jnp.dot(x_ref[...], w1_ref[0], preferred_element_type=jnp.float32)
        h = jnp.maximum(h + b1_ref[0], 0.0)
        o_ref[...] = (jnp.dot(h, w2_ref[0], preferred_element_type=jnp.float32)
                      + b2_ref[0])


def _grouped_mlp(x_pad, tile_eid, ntiles, W1, b1, W2, b2, g_max):
    grid_spec = pltpu.PrefetchScalarGridSpec(
        num_scalar_prefetch=2,
        grid=(g_max,),
        in_specs=[
            pl.BlockSpec((_T, _D_MODEL), lambda g, e, m: (g, 0)),
            pl.BlockSpec((1, _D_MODEL, _D_FF), lambda g, e, m: (e[g], 0, 0)),
            pl.BlockSpec((1, 1, _D_FF), lambda g, e, m: (e[g], 0, 0)),
            pl.BlockSpec((1, _D_FF, _D_MODEL), lambda g, e, m: (e[g], 0, 0)),
            pl.BlockSpec((1, 1, _D_MODEL), lambda g, e, m: (e[g], 0, 0)),
        ],
        out_specs=pl.BlockSpec((_T, _D_MODEL), lambda g, e, m: (g, 0)),
    )
    return pl.pallas_call(
        _moe_body,
        grid_spec=grid_spec,
        out_shape=jax.ShapeDtypeStruct((g_max * _T, _D_MODEL), jnp.float32),
        compiler_params=pltpu.CompilerParams(
            dimension_semantics=("arbitrary",),
        ),
    )(tile_eid, ntiles, x_pad, W1, b1.reshape(_E, 1, _D_FF), W2,
      b2.reshape(_E, 1, _D_MODEL))


def kernel(inputs, Wg, W1, b1, W2, b2):
    flat = inputs.reshape((-1, inputs.shape[-1]))
    n = flat.shape[0]
    g_max = -(-n // _T) + _E - 1

    pos, meta = _route(flat, Wg, g_max)
    tile_eid = meta[:g_max]
    ntiles = meta[g_max:g_max + 1]

    x_pad = _sc_row_scatter(flat, pos, g_max * _T, _D_MODEL)
    y_pad = _grouped_mlp(x_pad, tile_eid, ntiles, W1, b1, W2, b2, g_max)
    out = _sc_row_gather(y_pad, pos, n, _D_MODEL)
    return out.reshape(inputs.shape[:-1] + (_D_MODEL,))


# speedup vs baseline: 3.6956x; 1.0529x over previous
import functools
import jax
import jax.numpy as jnp
from jax import lax
from jax.experimental import pallas as pl
from jax.experimental.pallas import tpu as pltpu
from jax.experimental.pallas import tpu_sc as plsc

_E = 16
_D_MODEL = 768
_D_FF = 2048
_T = 160


def _sc_info():
    info = plsc.get_sparse_core_info()
    return info.num_cores, info.num_subcores


def _sc_row_gather(table, idx, n_rows, d):
    nc, ns = _sc_info()
    nw = nc * ns
    b_per_w = n_rows // nw
    chunk = 64
    n_chunks = b_per_w // chunk
    assert chunk * n_chunks == b_per_w
    nbuf = min(2, n_chunks)
    mesh = plsc.VectorSubcoreMesh(core_axis_name="c", subcore_axis_name="s")

    @functools.partial(
        pl.kernel,
        mesh=mesh,
        out_type=jax.ShapeDtypeStruct((n_rows, d), jnp.float32),
        scratch_types=[pltpu.VMEM((chunk,), jnp.int32) for _ in range(n_chunks)]
        + [pltpu.VMEM((chunk, d), jnp.float32) for _ in range(nbuf)]
        + [pltpu.SemaphoreType.DMA for _ in range(nbuf)],
    )
    def k(table_hbm, idx_hbm, out_hbm, *rest):
        idx_bufs = rest[:n_chunks]
        rows_bufs = rest[n_chunks:n_chunks + nbuf]
        sems = rest[n_chunks + nbuf:]
        wid = lax.axis_index("s") * nc + lax.axis_index("c")
        base = wid * b_per_w
        for c in range(n_chunks):
            pltpu.sync_copy(idx_hbm.at[pl.ds(base + c * chunk, chunk)],
                            idx_bufs[c])
        copies = [None] * n_chunks
        for c in range(min(nbuf, n_chunks)):
            copies[c] = pltpu.async_copy(table_hbm.at[idx_bufs[c]],
                                         rows_bufs[c % nbuf], sems[c % nbuf])
        for c in range(n_chunks):
            copies[c].wait()
            pltpu.sync_copy(rows_bufs[c % nbuf],
                            out_hbm.at[pl.ds(base + c * chunk, chunk)])
            nxt = c + nbuf
            if nxt < n_chunks:
                copies[nxt] = pltpu.async_copy(table_hbm.at[idx_bufs[nxt]],
                                               rows_bufs[nxt % nbuf],
                                               sems[nxt % nbuf])

    return k(table, idx)


def _sc_row_scatter(rows, idx, n_out_rows, d):
    nc, ns = _sc_info()
    nw = nc * ns
    n_in = rows.shape[0]
    b_per_w = n_in // nw
    chunk = 64
    n_chunks = b_per_w // chunk
    assert chunk * n_chunks == b_per_w
    mesh = plsc.VectorSubcoreMesh(core_axis_name="c", subcore_axis_name="s")

    @functools.partial(
        pl.kernel,
        mesh=mesh,
        out_type=jax.ShapeDtypeStruct((n_out_rows, d), jnp.float32),
        scratch_types=[pltpu.VMEM((chunk,), jnp.int32) for _ in range(n_chunks)]
        + [pltpu.VMEM((chunk, d), jnp.float32) for _ in range(n_chunks)]
        + [pltpu.SemaphoreType.DMA],
    )
    def k(rows_hbm, idx_hbm, out_hbm, *rest):
        idx_bufs = rest[:n_chunks]
        rows_bufs = rest[n_chunks:2 * n_chunks]
        sem = rest[2 * n_chunks]
        wid = lax.axis_index("s") * nc + lax.axis_index("c")
        base = wid * b_per_w
        for c in range(n_chunks):
            pltpu.sync_copy(idx_hbm.at[pl.ds(base + c * chunk, chunk)],
                            idx_bufs[c])
            pltpu.sync_copy(rows_hbm.at[pl.ds(base + c * chunk, chunk)],
                            rows_bufs[c])
        copies = [pltpu.async_copy(rows_bufs[c], out_hbm.at[idx_bufs[c]], sem)
                  for c in range(n_chunks)]
        for cp in copies:
            cp.wait()

    return k(rows, idx)


def _route_body(x_ref, wg_ref, pos_ref, meta_ref):
    n = x_ref.shape[0]
    g_max = meta_ref.shape[0] - 8
    logits = jnp.dot(x_ref[...], wg_ref[...], preferred_element_type=jnp.float32)
    m = jnp.max(logits, axis=-1, keepdims=True)
    s = jnp.exp(logits - m)
    prob = s / jnp.sum(s, axis=-1, keepdims=True)
    eid = jnp.argmax(prob, axis=-1).astype(jnp.int32)

    oh = (eid[:, None] == lax.broadcasted_iota(jnp.int32, (1, _E), 1)
          ).astype(jnp.int32)
    cum = oh
    sh = 1
    while sh < n:
        cum = cum + jnp.concatenate(
            [jnp.zeros((sh, _E), jnp.int32), cum[:-sh]], axis=0)
        sh *= 2
    counts_i = cum[-1:]
    counts = counts_i.astype(jnp.float32)

    col = lax.broadcasted_iota(jnp.int32, (_E, _E), 1)
    row = lax.broadcasted_iota(jnp.int32, (_E, _E), 0)
    tri_excl = (row < col).astype(jnp.float32)
    tri_incl = (row <= col).astype(jnp.float32)

    start = jnp.dot(counts, tri_excl,
                    preferred_element_type=jnp.float32).astype(jnp.int32)
    tiles_e = ((counts_i + (_T - 1)) // _T).astype(jnp.float32)
    cum_tiles = jnp.dot(tiles_e, tri_incl,
                        preferred_element_type=jnp.float32).astype(jnp.int32)
    tile_off = cum_tiles - tiles_e.astype(jnp.int32)
    ntiles = cum_tiles[0, _E - 1]

    rank = jnp.sum(cum * oh, axis=1) - 1
    start_tok = jnp.sum(start * oh, axis=1)
    toff_tok = jnp.sum(tile_off * oh, axis=1)
    del start_tok
    pos_ref[...] = (toff_tok + rank // _T) * _T + rank % _T

    gi = lax.broadcasted_iota(jnp.int32, (g_max, _E), 0)
    tile_eid = jnp.minimum(
        jnp.sum((gi >= cum_tiles).astype(jnp.int32), axis=1), _E - 1)
    meta_ref[...] = jnp.concatenate(
        [tile_eid, jnp.full((8,), ntiles, jnp.int32)], axis=0)


def _route(flat, Wg, g_max):
    n = flat.shape[0]
    return pl.pallas_call(
        _route_body,
        out_shape=[jax.ShapeDtypeStruct((n,), jnp.int32),
                   jax.ShapeDtypeStruct((g_max + 8,), jnp.int32)],
    )(flat, Wg)


def _moe_body(eids_ref, meta_ref, x_ref, w1_ref, b1_ref, w2_ref, b2_ref, o_ref):
    g = pl.program_id(0)

    @pl.when(g < meta_ref[0])
    def _():
        h = jnp.dot(x_ref[...], w1_ref[0], preferred_element_type=jnp.float32)
        h = jnp.maximum(h + b1_ref[0], 0.0)
        o_ref[...] = (jnp.dot(h, w2_ref[0], preferred_element_type=jnp.float32)
                      + b2_ref[0])


def _grouped_mlp(x_pad, tile_eid, ntiles, W1, b1, W2, b2, g_max):
    grid_spec = pltpu.PrefetchScalarGridSpec(
        num_scalar_prefetch=2,
        grid=(g_max,),
        in_specs=[
            pl.BlockSpec((_T, _D_MODEL),
                         lambda g, e, m: (jnp.minimum(g, m[0] - 1), 0)),
            pl.BlockSpec((1, _D_MODEL, _D_FF), lambda g, e, m: (e[g], 0, 0)),
            pl.BlockSpec((1, 1, _D_FF), lambda g, e, m: (e[g], 0, 0)),
            pl.BlockSpec((1, _D_FF, _D_MODEL), lambda g, e, m: (e[g], 0, 0)),
            pl.BlockSpec((1, 1, _D_MODEL), lambda g, e, m: (e[g], 0, 0)),
        ],
        out_specs=pl.BlockSpec((_T, _D_MODEL),
                               lambda g, e, m: (jnp.minimum(g, m[0] - 1), 0)),
    )
    return pl.pallas_call(
        _moe_body,
        grid_spec=grid_spec,
        out_shape=jax.ShapeDtypeStruct((g_max * _T, _D_MODEL), jnp.float32),
        compiler_params=pltpu.CompilerParams(
            dimension_semantics=("arbitrary",),
        ),
    )(tile_eid, ntiles, x_pad, W1, b1.reshape(_E, 1, _D_FF), W2,
      b2.reshape(_E, 1, _D_MODEL))


def kernel(inputs, Wg, W1, b1, W2, b2):
    flat = inputs.reshape((-1, inputs.shape[-1]))
    n = flat.shape[0]
    g_max = -(-n // _T) + _E - 1

    pos, meta = _route(flat, Wg, g_max)
    tile_eid = meta[:g_max]
    ntiles = meta[g_max:g_max + 1]

    x_pad = _sc_row_scatter(flat, pos, g_max * _T, _D_MODEL)
    y_pad = _grouped_mlp(x_pad, tile_eid, ntiles, W1, b1, W2, b2, g_max)
    out = _sc_row_gather(y_pad, pos, n, _D_MODEL)
    return out.reshape(inputs.shape[:-1] + (_D_MODEL,))


# speedup vs baseline: 3.7435x; 1.0130x over previous
import functools
import jax
import jax.numpy as jnp
from jax import lax
from jax.experimental import pallas as pl
from jax.experimental.pallas import tpu as pltpu
from jax.experimental.pallas import tpu_sc as plsc

_E = 16
_D_MODEL = 768
_D_FF = 2048
_T = 160


def _sc_info():
    info = plsc.get_sparse_core_info()
    return info.num_cores, info.num_subcores


def _sc_row_gather(table, idx, n_rows, d):
    nc, ns = _sc_info()
    nw = nc * ns
    b_per_w = n_rows // nw
    chunk = 64
    n_chunks = b_per_w // chunk
    assert chunk * n_chunks == b_per_w
    nbuf = min(2, n_chunks)
    mesh = plsc.VectorSubcoreMesh(core_axis_name="c", subcore_axis_name="s")

    @functools.partial(
        pl.kernel,
        mesh=mesh,
        out_type=jax.ShapeDtypeStruct((n_rows, d), jnp.float32),
        scratch_types=[pltpu.VMEM((chunk,), jnp.int32) for _ in range(n_chunks)]
        + [pltpu.VMEM((chunk, d), jnp.float32) for _ in range(nbuf)]
        + [pltpu.SemaphoreType.DMA for _ in range(nbuf)],
    )
    def k(table_hbm, idx_hbm, out_hbm, *rest):
        idx_bufs = rest[:n_chunks]
        rows_bufs = rest[n_chunks:n_chunks + nbuf]
        sems = rest[n_chunks + nbuf:]
        wid = lax.axis_index("s") * nc + lax.axis_index("c")
        base = wid * b_per_w
        for c in range(n_chunks):
            pltpu.sync_copy(idx_hbm.at[pl.ds(base + c * chunk, chunk)],
                            idx_bufs[c])
        copies = [None] * n_chunks
        for c in range(min(nbuf, n_chunks)):
            copies[c] = pltpu.async_copy(table_hbm.at[idx_bufs[c]],
                                         rows_bufs[c % nbuf], sems[c % nbuf])
        for c in range(n_chunks):
            copies[c].wait()
            pltpu.sync_copy(rows_bufs[c % nbuf],
                            out_hbm.at[pl.ds(base + c * chunk, chunk)])
            nxt = c + nbuf
            if nxt < n_chunks:
                copies[nxt] = pltpu.async_copy(table_hbm.at[idx_bufs[nxt]],
                                               rows_bufs[nxt % nbuf],
                                               sems[nxt % nbuf])

    return k(table, idx)


def _sc_row_scatter(rows, idx, n_out_rows, d):
    nc, ns = _sc_info()
    nw = nc * ns
    n_in = rows.shape[0]
    b_per_w = n_in // nw
    chunk = 64
    n_chunks = b_per_w // chunk
    assert chunk * n_chunks == b_per_w
    mesh = plsc.VectorSubcoreMesh(core_axis_name="c", subcore_axis_name="s")

    @functools.partial(
        pl.kernel,
        mesh=mesh,
        out_type=jax.ShapeDtypeStruct((n_out_rows, d), jnp.float32),
        scratch_types=[pltpu.VMEM((chunk,), jnp.int32) for _ in range(n_chunks)]
        + [pltpu.VMEM((chunk, d), jnp.float32) for _ in range(n_chunks)]
        + [pltpu.SemaphoreType.DMA],
    )
    def k(rows_hbm, idx_hbm, out_hbm, *rest):
        idx_bufs = rest[:n_chunks]
        rows_bufs = rest[n_chunks:2 * n_chunks]
        sem = rest[2 * n_chunks]
        wid = lax.axis_index("s") * nc + lax.axis_index("c")
        base = wid * b_per_w
        for c in range(n_chunks):
            pltpu.sync_copy(idx_hbm.at[pl.ds(base + c * chunk, chunk)],
                            idx_bufs[c])
            pltpu.sync_copy(rows_hbm.at[pl.ds(base + c * chunk, chunk)],
                            rows_bufs[c])
        copies = [pltpu.async_copy(rows_bufs[c], out_hbm.at[idx_bufs[c]], sem)
                  for c in range(n_chunks)]
        for cp in copies:
            cp.wait()

    return k(rows, idx)


def _route_body(x_ref, wg_ref, pos_ref, meta_ref):
    n = x_ref.shape[0]
    g_max = meta_ref.shape[0] - 8
    logits = jnp.dot(x_ref[...], wg_ref[...], preferred_element_type=jnp.float32)
    m = jnp.max(logits, axis=-1, keepdims=True)
    s = jnp.exp(logits - m)
    prob = s / jnp.sum(s, axis=-1, keepdims=True)
    eid = jnp.argmax(prob, axis=-1).astype(jnp.int32)

    oh = (eid[:, None] == lax.broadcasted_iota(jnp.int32, (1, _E), 1)
          ).astype(jnp.int32)
    cum = oh
    sh = 1
    while sh < n:
        cum = cum + jnp.concatenate(
            [jnp.zeros((sh, _E), jnp.int32), cum[:-sh]], axis=0)
        sh *= 2
    counts_i = cum[-1:]
    counts = counts_i.astype(jnp.float32)

    col = lax.broadcasted_iota(jnp.int32, (_E, _E), 1)
    row = lax.broadcasted_iota(jnp.int32, (_E, _E), 0)
    tri_incl = (row <= col).astype(jnp.float32)

    tiles_e = ((counts_i + (_T - 1)) // _T).astype(jnp.float32)
    cum_tiles = jnp.dot(tiles_e, tri_incl,
                        preferred_element_type=jnp.float32).astype(jnp.int32)
    tile_off = cum_tiles - tiles_e.astype(jnp.int32)
    ntiles = cum_tiles[0, _E - 1]

    rank = jnp.sum(cum * oh, axis=1) - 1
    toff_tok = jnp.sum(tile_off * oh, axis=1)
    pos_ref[...] = (toff_tok + rank // _T) * _T + rank % _T

    gi = lax.broadcasted_iota(jnp.int32, (g_max, _E), 0)
    tile_eid = jnp.minimum(
        jnp.sum((gi >= cum_tiles).astype(jnp.int32), axis=1), _E - 1)
    meta_ref[...] = jnp.concatenate(
        [tile_eid, jnp.full((8,), ntiles, jnp.int32)], axis=0)


def _route(flat, Wg, g_max):
    n = flat.shape[0]
    return pl.pallas_call(
        _route_body,
        out_shape=[jax.ShapeDtypeStruct((n,), jnp.int32),
                   jax.ShapeDtypeStruct((g_max + 8,), jnp.int32)],
    )(flat, Wg)


def _moe_body(eids_ref, meta_ref, x_ref, w1_ref, b1_ref, w2_ref, b2_ref, o_ref):
    g = pl.program_id(0)

    @pl.when(g < meta_ref[0])
    def _():
        h = jnp.dot(x_ref[...], w1_ref[0], preferred_element_type=jnp.float32)
        h = jnp.maximum(h + b1_ref[0], 0.0)
        o_ref[...] = (jnp.dot(h, w2_ref[0], preferred_element_type=jnp.float32)
                      + b2_ref[0])


def _grouped_mlp(x_pad, tile_eid, ntiles, W1, b1, W2, b2, g_max):
    grid_spec = pltpu.PrefetchScalarGridSpec(
        num_scalar_prefetch=2,
        grid=(g_max,),
        in_specs=[
            pl.BlockSpec((_T, _D_MODEL),
                         lambda g, e, m: (jnp.minimum(g, m[0] - 1), 0)),
            pl.BlockSpec((1, _D_MODEL, _D_FF), lambda g, e, m: (e[g], 0, 0)),
            pl.BlockSpec((1, 1, _D_FF), lambda g, e, m: (e[g], 0, 0)),
            pl.BlockSpec((1, _D_FF, _D_MODEL), lambda g, e, m: (e[g], 0, 0)),
            pl.BlockSpec((1, 1, _D_MODEL), lambda g, e, m: (e[g], 0, 0)),
        ],
        out_specs=pl.BlockSpec((_T, _D_MODEL),
                               lambda g, e, m: (jnp.minimum(g, m[0] - 1), 0)),
    )
    return pl.pallas_call(
        _moe_body,
        grid_spec=grid_spec,
        out_shape=jax.ShapeDtypeStruct((g_max * _T, _D_MODEL), jnp.float32),
        compiler_params=pltpu.CompilerParams(
            dimension_semantics=("arbitrary",),
        ),
    )(tile_eid, ntiles, x_pad, W1, b1.reshape(_E, 1, _D_FF), W2,
      b2.reshape(_E, 1, _D_MODEL))


def kernel(inputs, Wg, W1, b1, W2, b2):
    flat = inputs.reshape((-1, inputs.shape[-1]))
    n = flat.shape[0]
    g_max = -(-n // _T) + _E - 1

    pos, meta = _route(flat, Wg, g_max)
    tile_eid = meta[:g_max]
    ntiles = meta[g_max:g_max + 1]

    x_pad = _sc_row_scatter(flat, pos, g_max * _T, _D_MODEL)
    y_pad = _grouped_mlp(x_pad, tile_eid, ntiles, W1, b1, W2, b2, g_max)
    out = _sc_row_gather(y_pad, pos, n, _D_MODEL)
    return out.reshape(inputs.shape[:-1] + (_D_MODEL,))
